# baseline (speedup 1.0000x reference)
import functools

import jax
import jax.numpy as jnp
from jax import lax
from jax.experimental import pallas as pl
from jax.experimental.pallas import tpu as pltpu
from jax.experimental.pallas import tpu_sc as plsc

_LANES = 16
_KTOK = 128
_BPC = 33


@functools.partial(jax.jit, static_argnums=(3, 4, 5))
def _sc_embed(idx3, token_table, comb, T, D, NW):
  NB = idx3.shape[0]
  G = NB // NW
  mesh = plsc.VectorSubcoreMesh(core_axis_name="c", subcore_axis_name="s")

  @functools.partial(
      pl.kernel,
      mesh=mesh,
      out_type=jax.ShapeDtypeStruct((T + _KTOK, D), jnp.float32),
      scratch_types=[
          pltpu.VMEM((4, 3, _KTOK), jnp.int32),
          pltpu.VMEM((2, _KTOK, D), jnp.float32),
          pltpu.VMEM((2, 1, D), jnp.float32),
      ] + [pltpu.SemaphoreType.DMA] * 10,
  )
  def k(idx_hbm, tab_hbm, comb_hbm, out_hbm, idx_v, rows_v, crow_v,
        s_i0, s_i1, s_i2, s_i3, s_g0, s_g1, s_c0, s_c1, s_o0, s_o1):
    sem_i = (s_i0, s_i1, s_i2, s_i3)
    sem_g = (s_g0, s_g1)
    sem_c = (s_c0, s_c1)
    sem_o = (s_o0, s_o1)
    wid = lax.axis_index("s") * 2 + lax.axis_index("c")
    row0 = wid * G

    def issue_idx(r, u):
      pltpu.async_copy(idx_hbm.at[r], idx_v.at[u], sem_i[u])

    def wait_idx(u):
      pltpu.make_async_copy(idx_hbm.at[0], idx_v.at[u], sem_i[u]).wait()

    def issue_gath(p, u):
      pltpu.async_copy(tab_hbm.at[idx_v.at[u, 0]], rows_v.at[p], sem_g[p])
      pltpu.async_copy(comb_hbm.at[idx_v.at[u, 2, pl.ds(0, 1)]], crow_v.at[p],
                       sem_c[p])

    def wait_gath(p, u):
      pltpu.make_async_copy(tab_hbm.at[idx_v.at[u, 0]], rows_v.at[p],
                            sem_g[p]).wait()
      pltpu.make_async_copy(comb_hbm.at[idx_v.at[u, 2, pl.ds(0, 1)]],
                            crow_v.at[p], sem_c[p]).wait()

    def issue_out(p, u):
      pltpu.async_copy(rows_v.at[p], out_hbm.at[idx_v.at[u, 1]], sem_o[p])

    def wait_out(p):
      pltpu.make_async_copy(rows_v.at[p], out_hbm.at[idx_v.at[0, 1]],
                            sem_o[p]).wait()

    def compute(p):
      cvals = [crow_v[p, 0, pl.ds(j * _LANES, _LANES)]
               for j in range(D // _LANES)]

      def add_body(i, c_):
        for j in range(D // _LANES):
          plsc.addupdate(rows_v.at[p, i, pl.ds(j * _LANES, _LANES)], cvals[j])
        return c_

      lax.fori_loop(0, _KTOK, add_body, 0, unroll=4)

    issue_idx(row0, 0)
    issue_idx(row0 + 1, 1)
    issue_idx(row0 + 2, 2)
    wait_idx(0)
    issue_gath(0, 0)

    nT = G // 4

    def body(t, carry):
      for b in range(4):
        p = b % 2
        u = b
        un = (b + 1) % 4
        r = row0 + 4 * t + b

        wait_gath(p, u)

        def gadv():
          wait_idx(un)
          issue_gath(1 - p, un)

        if b == 0:
          @pl.when(t >= 1)
          def _():
            wait_out(1)

          gadv()
          issue_idx(r + 3, 3)
        elif b < 3:
          wait_out(1 - p)
          gadv()
          pl.when(t < nT - 1)(lambda: issue_idx(r + 3, (b + 3) % 4))
        else:
          def last_adv():
            wait_out(0)
            gadv()
            issue_idx(r + 3, 2)

          pl.when(t < nT - 1)(last_adv)

        compute(p)
        issue_out(p, u)
      return carry

    lax.fori_loop(0, nT, body, 0)
    wait_out(0)
    wait_out(1)

  return k(idx3, token_table, comb)


def _prep_blocks(x, segment_tokens, B, L, T, NB):
  slotc = _BPC * _KTOK
  sT = segment_tokens.T.astype(jnp.int32)
  xT = x.T.astype(jnp.int32)
  incl1 = jnp.cumsum(sT, axis=1)
  n1 = incl1[:, -1]
  excl1 = incl1 - sT
  barange = jnp.arange(B, dtype=jnp.int32)[None, :]
  excl0 = barange - excl1
  n0 = B - n1
  nb0 = (n0 + _KTOK - 1) // _KTOK
  slot = jnp.where(sT == 0, excl0, nb0[:, None] * _KTOK + excl1)
  larange = jnp.arange(L, dtype=jnp.int32)
  gslot = (larange[:, None] * slotc + slot).reshape(-1)
  opos = (barange * L + larange[:, None]).reshape(-1)
  npad = NB * _KTOK - L * slotc
  tok = jnp.zeros((L * slotc,), jnp.int32).at[gslot].set(xT.reshape(-1))
  out = jnp.full((L * slotc,), T, jnp.int32).at[gslot].set(opos)
  tok = jnp.concatenate([tok, jnp.zeros((npad,), jnp.int32)])
  out = jnp.concatenate([out, jnp.full((npad,), T, jnp.int32)])
  cblk = jnp.where(jnp.arange(_BPC, dtype=jnp.int32)[None, :] < nb0[:, None],
                   larange[:, None], larange[:, None] + L)
  cblk = jnp.concatenate(
      [cblk.reshape(-1), jnp.zeros((NB - L * _BPC,), jnp.int32)])
  comb3 = jnp.broadcast_to(cblk[:, None], (NB, _KTOK))
  return jnp.stack(
      [tok.reshape(NB, _KTOK), out.reshape(NB, _KTOK), comb3], axis=1)


def kernel(x, segment_tokens, token_table, segment_table, pe):
  B, L = x.shape
  V, D = token_table.shape
  T = B * L
  NW = 32
  NB = -(-(L * _BPC) // NW) * NW
  NB += (-NB) % (4 * NW)
  comb = (segment_table.astype(jnp.float32)[:, None, :]
          + pe[:L, :][None, :, :]).reshape(2 * L, D)
  idx3 = _prep_blocks(x, segment_tokens, B, L, T, NB)
  out = _sc_embed(idx3, token_table, comb, T, D, NW)
  return out[:T].reshape(B, L, D)


# speedup vs baseline: 5.8845x; 5.8845x over previous
import functools

import jax
import jax.numpy as jnp
from jax import lax
from jax.experimental import pallas as pl
from jax.experimental.pallas import tpu as pltpu
from jax.experimental.pallas import tpu_sc as plsc

_LANES = 16
_KTOK = 128


_IDXW = _KTOK + _KTOK * _LANES


@functools.partial(jax.jit, static_argnums=(5, 6, 7, 8))
def _sc_embed(idx2, msp, token_table, pe0, dif, T, D, L, NW):
  G = (T // _KTOK) // NW
  mesh = plsc.VectorSubcoreMesh(core_axis_name="c", subcore_axis_name="s")

  @functools.partial(
      pl.kernel,
      mesh=mesh,
      out_type=jax.ShapeDtypeStruct((T, D), jnp.float32),
      scratch_types=[
          pltpu.VMEM((2, 1, _KTOK), jnp.int32),
          pltpu.VMEM((2, _KTOK, _LANES), jnp.float32),
          pltpu.VMEM((2, _KTOK, D), jnp.float32),
          pltpu.VMEM((L, D), jnp.float32),
          pltpu.VMEM((1, D), jnp.float32),
      ] + [pltpu.SemaphoreType.DMA] * 8,
  )
  def k(idx_hbm, msp_hbm, tab_hbm, pe0_hbm, dif_hbm, out_hbm, idx_v, msp_v,
        rows_v, pe_v, dif_v, s_i0, s_i1, s_m0, s_m1, s_g0, s_g1, s_o0, s_o1):
    sem_i = (s_i0, s_i1)
    sem_m = (s_m0, s_m1)
    sem_g = (s_g0, s_g1)
    sem_o = (s_o0, s_o1)
    wid = lax.axis_index("s") * 2 + lax.axis_index("c")
    row0 = wid * G
    nj = D // _LANES

    def issue_idx(r, p):
      pltpu.async_copy(idx_hbm.at[r], idx_v.at[p], sem_i[p])
      pltpu.async_copy(msp_hbm.at[r], msp_v.at[p], sem_m[p])

    def wait_idx(p):
      pltpu.make_async_copy(idx_hbm.at[0], idx_v.at[p], sem_i[p]).wait()
      pltpu.make_async_copy(msp_hbm.at[0], msp_v.at[p], sem_m[p]).wait()

    def issue_gath(p):
      pltpu.async_copy(tab_hbm.at[idx_v.at[p, 0]], rows_v.at[p], sem_g[p])

    def wait_gath(p):
      pltpu.make_async_copy(tab_hbm.at[idx_v.at[p, 0]], rows_v.at[p],
                            sem_g[p]).wait()

    def issue_out(r, p):
      pltpu.async_copy(rows_v.at[p], out_hbm.at[pl.ds(r * _KTOK, _KTOK)],
                       sem_o[p])

    def wait_out(p):
      pltpu.make_async_copy(rows_v.at[p], out_hbm.at[pl.ds(0, _KTOK)],
                            sem_o[p]).wait()

    pltpu.sync_copy(pe0_hbm, pe_v)
    pltpu.sync_copy(dif_hbm, dif_v)
    dvals = [dif_v[0, pl.ds(j * _LANES, _LANES)] for j in range(nj)]

    def compute(p, r):
      li0 = lax.rem(r * _KTOK, L)

      def tok_body(i, li):
        m = msp_v[p, i]
        for j in range(nj):
          val = pe_v[li, pl.ds(j * _LANES, _LANES)] + m * dvals[j]
          plsc.addupdate(rows_v.at[p, i, pl.ds(j * _LANES, _LANES)], val)
        return jnp.where(li == L - 1, 0, li + 1)

      lax.fori_loop(0, _KTOK, tok_body, li0, unroll=2)

    issue_idx(row0, 0)
    issue_idx(row0 + 1, 1)
    wait_idx(0)
    issue_gath(0)

    def body(t, carry):
      for b in range(2):
        p = b
        q = 1 - b
        r = row0 + 2 * t + b
        wait_gath(p)
        if b == 0:
          wait_idx(q)

          @pl.when(t >= 1)
          def _():
            wait_out(q)

          issue_gath(q)
        else:
          @pl.when(t < G // 2 - 1)
          def _():
            wait_idx(q)
            wait_out(q)
            issue_gath(q)
        compute(p, r)
        issue_out(r, p)

        @pl.when(t < G // 2 - 1)
        def _():
          issue_idx(r + 2, p)
      return carry

    lax.fori_loop(0, G // 2, body, 0)
    wait_out(0)
    wait_out(1)

  return k(idx2, msp, token_table, pe0, dif)


def kernel(x, segment_tokens, token_table, segment_table, pe):
  B, L = x.shape
  V, D = token_table.shape
  T = B * L
  NW = 32
  seg = segment_table.astype(jnp.float32)
  pe0 = pe[:L, :] + seg[0][None, :]
  dif = (seg[1] - seg[0]).reshape(1, D)
  NBLK = T // _KTOK
  idx2 = x.astype(jnp.int32).reshape(NBLK, 1, _KTOK)
  m2 = segment_tokens.astype(jnp.float32).reshape(NBLK, _KTOK)
  msp = jnp.broadcast_to(m2[:, :, None], (NBLK, _KTOK, _LANES))
  out = _sc_embed(idx2, msp, token_table, pe0, dif, T, D, L, NW)
  return out.reshape(B, L, D)
